# Initial kernel scaffold; baseline (speedup 1.0000x reference)
import functools

import jax
import jax.numpy as jnp
from jax import lax
from jax.experimental import pallas as pl
from jax.experimental.pallas import tpu as pltpu
from jax.experimental.pallas import tpu_sc as plsc

N = 50000
D = 128
E = 600000
NC = 2
NS = 16
CHUNKS = 4
CW = 32
GROUP = 128
RPT = 293
EP = NS * RPT * GROUP
ACC_N = 50048
ROWS_T = ACC_N // NS
WBR = 391
BN = 400


def _mm_body(fu_ref, fi_ref, w_ref, b_ref, tab_ref):
    for e in range(2):
        f = fu_ref[...] if e == 0 else fi_ref[...]
        wh = jnp.dot(f, w_ref[e], preferred_element_type=jnp.float32)
        wh = wh + b_ref[e][None, :]
        for q in range(CHUNKS):
            tab_ref[e, q] = wh[:, q * CW:(q + 1) * CW]


def _make_tables(feat_user, feat_item, ws, bs):
    return pl.pallas_call(
        _mm_body,
        grid=(N // BN,),
        in_specs=[
            pl.BlockSpec((BN, D), lambda i: (i, 0)),
            pl.BlockSpec((BN, D), lambda i: (i, 0)),
            pl.BlockSpec((2, D, D), lambda i: (0, 0, 0)),
            pl.BlockSpec((2, D), lambda i: (0, 0)),
        ],
        out_specs=pl.BlockSpec((2, CHUNKS, BN, CW), lambda i: (0, 0, i, 0)),
        out_shape=jax.ShapeDtypeStruct((2, CHUNKS, N, CW), jnp.float32),
    )(feat_user, feat_item, ws, bs)


_MESH = plsc.VectorSubcoreMesh(core_axis_name="c", subcore_axis_name="s")


@functools.partial(
    pl.kernel,
    out_type=(
        jax.ShapeDtypeStruct((2, CHUNKS, ACC_N, CW), jnp.float32),
        jax.ShapeDtypeStruct((2, ACC_N, CW), jnp.float32),
    ),
    mesh=_MESH,
    scratch_types=[
        pltpu.VMEM((RPT, GROUP), jnp.int32),
        pltpu.VMEM((RPT, GROUP), jnp.int32),
        pltpu.VMEM((GROUP, CW), jnp.float32),
        pltpu.VMEM((GROUP, CW), jnp.float32),
        pltpu.VMEM((WBR, CW), jnp.float32),
        pltpu.VMEM((WBR, CW), jnp.float32),
        pltpu.VMEM_SHARED((ACC_N, CW), jnp.float32),
        pltpu.SemaphoreType.DMA,
    ],
)
def _sc_segment_sums(tabs, srcs, dsts, zeros_in, ones_in,
                     sums_out, cnt_out,
                     src_idx, dst_idx, rows_v, ones_v, zeros_v, wb_v,
                     acc, sem):
    cid = lax.axis_index("c")
    sid = lax.axis_index("s")
    r0 = sid * RPT
    a0 = sid * ROWS_T

    pltpu.sync_copy(srcs.at[cid, pl.ds(r0, RPT)], src_idx)
    pltpu.sync_copy(dsts.at[cid, pl.ds(r0, RPT)], dst_idx)
    pltpu.sync_copy(ones_in, ones_v)
    pltpu.sync_copy(zeros_in, zeros_v)

    def zero_acc():
        for k in range(ROWS_T // WBR):
            pltpu.sync_copy(zeros_v, acc.at[pl.ds(a0 + k * WBR, WBR)])

    def writeback(dst_ref):
        for k in range(ROWS_T // WBR):
            pltpu.sync_copy(acc.at[pl.ds(a0 + k * WBR, WBR)], wb_v)
            pltpu.sync_copy(wb_v, dst_ref.at[pl.ds(a0 + k * WBR, WBR)])

    zero_acc()
    plsc.subcore_barrier()

    def cnt_body(g, carry):
        pltpu.sync_copy(ones_v, acc.at[dst_idx.at[g]], add=True)
        return carry

    lax.fori_loop(0, RPT, cnt_body, 0)
    plsc.subcore_barrier()
    writeback(cnt_out.at[cid])

    for p in range(CHUNKS):
        zero_acc()
        plsc.subcore_barrier()
        tab2d = tabs.at[cid, p]

        def body(g, carry):
            pltpu.async_copy(tab2d.at[src_idx.at[g]], rows_v, sem).wait()
            pltpu.sync_copy(rows_v, acc.at[dst_idx.at[g]], add=True)
            return carry

        lax.fori_loop(0, RPT, body, 0)
        plsc.subcore_barrier()
        writeback(sums_out.at[cid, p])


def _div_body(sums_ref, cnt_ref, hu_ref, hi_ref):
    outs = []
    for e in range(2):
        c = jnp.maximum(cnt_ref[e, :, 0:1], 1.0)
        h = jnp.concatenate([sums_ref[e, q] for q in range(CHUNKS)], axis=-1)
        outs.append(h / c)
    hi_ref[...] = outs[0]
    hu_ref[...] = outs[1]


def _divide(sums, cnt):
    return pl.pallas_call(
        _div_body,
        grid=(N // BN,),
        in_specs=[
            pl.BlockSpec((2, CHUNKS, BN, CW), lambda i: (0, 0, i, 0)),
            pl.BlockSpec((2, BN, CW), lambda i: (0, i, 0)),
        ],
        out_specs=[
            pl.BlockSpec((BN, D), lambda i: (i, 0)),
            pl.BlockSpec((BN, D), lambda i: (i, 0)),
        ],
        out_shape=[
            jax.ShapeDtypeStruct((N, D), jnp.float32),
            jax.ShapeDtypeStruct((N, D), jnp.float32),
        ],
    )(sums, cnt)


def kernel(feat_user, feat_item, W_rates, b_rates, W_rated_by, b_rated_by,
           edge_index_rates, edge_index_rated_by):
    ws = jnp.stack([W_rates, W_rated_by])
    bs = jnp.stack([b_rates, b_rated_by])
    pad_src = jnp.zeros((EP - E,), jnp.int32)
    pad_dst = jnp.full((EP - E,), N, jnp.int32)
    srcs = jnp.stack([
        jnp.concatenate([edge_index_rates[0], pad_src]),
        jnp.concatenate([edge_index_rated_by[0], pad_src]),
    ]).reshape(2, NS * RPT, GROUP)
    dsts = jnp.stack([
        jnp.concatenate([edge_index_rates[1], pad_dst]),
        jnp.concatenate([edge_index_rated_by[1], pad_dst]),
    ]).reshape(2, NS * RPT, GROUP)
    zeros_in = jnp.zeros((WBR, CW), jnp.float32)
    ones_in = jnp.ones((GROUP, CW), jnp.float32)

    tabs = _make_tables(feat_user, feat_item, ws, bs)
    sums, cnt = _sc_segment_sums(tabs, srcs, dsts, zeros_in, ones_in)
    h_user, h_item = _divide(sums, cnt)
    return (h_user, h_item)


# speedup vs baseline: 2.4017x; 2.4017x over previous
import functools

import jax
import jax.numpy as jnp
from jax import lax
from jax.experimental import pallas as pl
from jax.experimental.pallas import tpu as pltpu
from jax.experimental.pallas import tpu_sc as plsc

N = 50000
D = 128
E = 600000
NC = 2
NS = 16
CHUNKS = 8
CW = 16
GROUP = 128
RPT = 296
EP = NS * RPT * GROUP
ACC_N = 50176
ROWS_T = ACC_N // NS
WBR = 448
BN = 400
IDXB = 8


def _mm_body(fu_ref, fi_ref, w_ref, b_ref, tab_ref):
    for e in range(2):
        f = fu_ref[...] if e == 0 else fi_ref[...]
        wh = jnp.dot(f, w_ref[e], preferred_element_type=jnp.float32)
        wh = wh + b_ref[e][None, :]
        for q in range(CHUNKS):
            tab_ref[e, q] = wh[:, q * CW:(q + 1) * CW]


def _make_tables(feat_user, feat_item, ws, bs):
    return pl.pallas_call(
        _mm_body,
        grid=(N // BN,),
        in_specs=[
            pl.BlockSpec((BN, D), lambda i: (i, 0)),
            pl.BlockSpec((BN, D), lambda i: (i, 0)),
            pl.BlockSpec((2, D, D), lambda i: (0, 0, 0)),
            pl.BlockSpec((2, D), lambda i: (0, 0)),
        ],
        out_specs=pl.BlockSpec((2, CHUNKS, BN, CW), lambda i: (0, 0, i, 0)),
        out_shape=jax.ShapeDtypeStruct((2, CHUNKS, N, CW), jnp.float32),
    )(feat_user, feat_item, ws, bs)


_MESH = plsc.VectorSubcoreMesh(core_axis_name="c", subcore_axis_name="s")


@functools.partial(
    pl.kernel,
    out_type=(
        jax.ShapeDtypeStruct((2, CHUNKS, ACC_N, CW), jnp.float32),
        jax.ShapeDtypeStruct((2, ACC_N, CW), jnp.float32),
    ),
    mesh=_MESH,
    compiler_params=pltpu.CompilerParams(use_tc_tiling_on_sc=False),
    scratch_types=[
        pltpu.VMEM((IDXB, GROUP), jnp.int32),
        pltpu.VMEM((IDXB, GROUP), jnp.int32),
        pltpu.VMEM((GROUP, CW), jnp.float32),
        pltpu.VMEM((GROUP, CW), jnp.float32),
        pltpu.VMEM((WBR, CW), jnp.float32),
        pltpu.VMEM((WBR, CW), jnp.float32),
        pltpu.VMEM_SHARED((ACC_N, CW), jnp.float32),
        pltpu.SemaphoreType.DMA,
    ],
)
def _sc_segment_sums(tabs, srcs, dsts, zeros_in, ones_in,
                     sums_out, cnt_out,
                     src_idx, dst_idx, rows_v, ones_v, zeros_v, wb_v,
                     acc, sem):
    cid = lax.axis_index("c")
    sid = lax.axis_index("s")
    r0 = sid * RPT
    a0 = sid * ROWS_T

    pltpu.sync_copy(ones_in, ones_v)
    pltpu.sync_copy(zeros_in, zeros_v)

    def zero_acc():
        for k in range(ROWS_T // WBR):
            pltpu.sync_copy(zeros_v, acc.at[pl.ds(a0 + k * WBR, WBR)])

    def writeback(dst_ref):
        for k in range(ROWS_T // WBR):
            pltpu.sync_copy(acc.at[pl.ds(a0 + k * WBR, WBR)], wb_v)
            pltpu.sync_copy(wb_v, dst_ref.at[pl.ds(a0 + k * WBR, WBR)])

    zero_acc()
    plsc.subcore_barrier()

    def cnt_blk(b, carry):
        pltpu.sync_copy(dsts.at[cid, pl.ds(r0 + b * IDXB, IDXB)], dst_idx)

        def cnt_body(g, carry2):
            pltpu.sync_copy(ones_v, acc.at[dst_idx.at[g]], add=True)
            return carry2

        lax.fori_loop(0, IDXB, cnt_body, 0)
        return carry

    lax.fori_loop(0, RPT // IDXB, cnt_blk, 0)
    plsc.subcore_barrier()
    writeback(cnt_out.at[cid])

    def feat_pass(p, carry):
        zero_acc()
        plsc.subcore_barrier()

        def blk(b, carry2):
            pltpu.sync_copy(srcs.at[cid, pl.ds(r0 + b * IDXB, IDXB)], src_idx)
            pltpu.sync_copy(dsts.at[cid, pl.ds(r0 + b * IDXB, IDXB)], dst_idx)

            def body(g, carry3):
                pltpu.async_copy(tabs.at[cid, p].at[src_idx.at[g]], rows_v,
                                 sem).wait()
                pltpu.sync_copy(rows_v, acc.at[dst_idx.at[g]], add=True)
                return carry3

            lax.fori_loop(0, IDXB, body, 0)
            return carry2

        lax.fori_loop(0, RPT // IDXB, blk, 0)
        plsc.subcore_barrier()
        writeback(sums_out.at[cid, p])
        return carry

    lax.fori_loop(0, CHUNKS, feat_pass, 0)


def _div_body(sums_ref, cnt_ref, hu_ref, hi_ref):
    outs = []
    for e in range(2):
        c = jnp.maximum(cnt_ref[e, :, 0:1], 1.0)
        h = jnp.concatenate([sums_ref[e, q] for q in range(CHUNKS)], axis=-1)
        outs.append(h / c)
    hi_ref[...] = outs[0]
    hu_ref[...] = outs[1]


def _divide(sums, cnt):
    return pl.pallas_call(
        _div_body,
        grid=(N // BN,),
        in_specs=[
            pl.BlockSpec((2, CHUNKS, BN, CW), lambda i: (0, 0, i, 0)),
            pl.BlockSpec((2, BN, CW), lambda i: (0, i, 0)),
        ],
        out_specs=[
            pl.BlockSpec((BN, D), lambda i: (i, 0)),
            pl.BlockSpec((BN, D), lambda i: (i, 0)),
        ],
        out_shape=[
            jax.ShapeDtypeStruct((N, D), jnp.float32),
            jax.ShapeDtypeStruct((N, D), jnp.float32),
        ],
    )(sums, cnt)


def kernel(feat_user, feat_item, W_rates, b_rates, W_rated_by, b_rated_by,
           edge_index_rates, edge_index_rated_by):
    ws = jnp.stack([W_rates, W_rated_by])
    bs = jnp.stack([b_rates, b_rated_by])
    pad_src = jnp.zeros((EP - E,), jnp.int32)
    pad_dst = jnp.full((EP - E,), N, jnp.int32)
    srcs = jnp.stack([
        jnp.concatenate([edge_index_rates[0], pad_src]),
        jnp.concatenate([edge_index_rated_by[0], pad_src]),
    ]).reshape(2, NS * RPT, GROUP)
    dsts = jnp.stack([
        jnp.concatenate([edge_index_rates[1], pad_dst]),
        jnp.concatenate([edge_index_rated_by[1], pad_dst]),
    ]).reshape(2, NS * RPT, GROUP)
    zeros_in = jnp.zeros((WBR, CW), jnp.float32)
    ones_in = jnp.ones((GROUP, CW), jnp.float32)

    tabs = _make_tables(feat_user, feat_item, ws, bs)
    sums, cnt = _sc_segment_sums(tabs, srcs, dsts, zeros_in, ones_in)
    h_user, h_item = _divide(sums, cnt)
    return (h_user, h_item)


# speedup vs baseline: 3.7070x; 1.5435x over previous
import functools

import jax
import jax.numpy as jnp
from jax import lax
from jax.experimental import pallas as pl
from jax.experimental.pallas import tpu as pltpu
from jax.experimental.pallas import tpu_sc as plsc

N = 50000
D = 128
E = 600000
NC = 2
NS = 16
CHUNKS = 8
CW = 16
GROUP = 128
RPT = 296
EP = NS * RPT * GROUP
ACC_N = 50176
ROWS_T = ACC_N // NS
WBR = 448
BN = 400
IDXB = 8
NB = RPT // IDXB
KQ = 4


def _mm_body(fu_ref, fi_ref, w_ref, b_ref, tab_ref):
    for e in range(2):
        f = fu_ref[...] if e == 0 else fi_ref[...]
        wh = jnp.dot(f, w_ref[e], preferred_element_type=jnp.float32)
        wh = wh + b_ref[e][None, :]
        for q in range(CHUNKS):
            tab_ref[e, q] = wh[:, q * CW:(q + 1) * CW]


def _make_tables(feat_user, feat_item, ws, bs):
    return pl.pallas_call(
        _mm_body,
        grid=(N // BN,),
        in_specs=[
            pl.BlockSpec((BN, D), lambda i: (i, 0)),
            pl.BlockSpec((BN, D), lambda i: (i, 0)),
            pl.BlockSpec((2, D, D), lambda i: (0, 0, 0)),
            pl.BlockSpec((2, D), lambda i: (0, 0)),
        ],
        out_specs=pl.BlockSpec((2, CHUNKS, BN, CW), lambda i: (0, 0, i, 0)),
        out_shape=jax.ShapeDtypeStruct((2, CHUNKS, N, CW), jnp.float32),
    )(feat_user, feat_item, ws, bs)


_MESH = plsc.VectorSubcoreMesh(core_axis_name="c", subcore_axis_name="s")


@functools.partial(
    pl.kernel,
    out_type=(
        jax.ShapeDtypeStruct((2, CHUNKS, ACC_N, CW), jnp.float32),
        jax.ShapeDtypeStruct((2, ACC_N, CW), jnp.float32),
    ),
    mesh=_MESH,
    compiler_params=pltpu.CompilerParams(use_tc_tiling_on_sc=False),
    scratch_types=[
        pltpu.VMEM((2, IDXB, GROUP), jnp.int32),
        pltpu.VMEM((2, IDXB, GROUP), jnp.int32),
        pltpu.VMEM((KQ, GROUP, CW), jnp.float32),
        pltpu.VMEM((KQ, GROUP, CW), jnp.float32),
        pltpu.VMEM((GROUP, CW), jnp.float32),
        pltpu.VMEM_SHARED((ACC_N, CW), jnp.float32),
        pltpu.SemaphoreType.DMA,
        pltpu.SemaphoreType.DMA,
    ],
)
def _sc_segment_sums(tabs, srcs, dsts, zslab, ones_in,
                     sums_out, cnt_out,
                     src_idx, dst_idx, rows_a, rows_b, ones_v,
                     acc, gsem, ssem):
    cid = lax.axis_index("c")
    sid = lax.axis_index("s")
    r0 = sid * RPT
    a0 = sid * ROWS_T
    acc_slab = acc.at[pl.ds(a0, ROWS_T)]

    pltpu.sync_copy(ones_in, ones_v)

    def load_src(b, par):
        pltpu.sync_copy(srcs.at[cid, pl.ds(r0 + b * IDXB, IDXB)],
                        src_idx.at[par])

    def load_dst(b, par):
        pltpu.sync_copy(dsts.at[cid, pl.ds(r0 + b * IDXB, IDXB)],
                        dst_idx.at[par])

    pltpu.sync_copy(zslab, acc_slab)
    plsc.subcore_barrier()
    load_dst(0, 0)

    def cnt_blk(b, carry):
        par = lax.rem(b, 2)
        opar = lax.rem(b + 1, 2)
        for j in range(IDXB):
            pltpu.async_copy(ones_v, acc.at[dst_idx.at[par, j]], ssem,
                             add=True)

        @pl.when(b > 0)
        def _():
            for j in range(IDXB):
                pltpu.make_async_copy(ones_v, acc.at[dst_idx.at[opar, j]],
                                      ssem).wait()

        @pl.when(b < NB - 1)
        def _():
            load_dst(b + 1, opar)
        return carry

    lax.fori_loop(0, NB, cnt_blk, 0)
    parl = lax.rem(NB - 1, 2)
    for j in range(IDXB):
        pltpu.make_async_copy(ones_v, acc.at[dst_idx.at[parl, j]],
                              ssem).wait()
    plsc.subcore_barrier()
    pltpu.sync_copy(acc_slab, cnt_out.at[cid].at[pl.ds(a0, ROWS_T)])

    def feat_pass(p, carry):
        pltpu.sync_copy(zslab, acc_slab)
        plsc.subcore_barrier()
        tab = tabs.at[cid, p]

        def g_copy(par, j, rowbuf, jj):
            return pltpu.make_async_copy(tab.at[src_idx.at[par, j]],
                                         rowbuf.at[jj], gsem)

        def s_copy(par, j, rowbuf, jj):
            return pltpu.make_async_copy(rowbuf.at[jj],
                                         acc.at[dst_idx.at[par, j]], ssem)

        load_src(0, 0)
        load_dst(0, 0)
        for jj in range(KQ):
            g_copy(0, jj, rows_a, jj).start()

        def blk(b, carry2):
            par = lax.rem(b, 2)
            npar = lax.rem(b + 1, 2)
            for jj in range(KQ):
                g_copy(par, jj, rows_a, jj).wait()

            @pl.when(b > 0)
            def _():
                for jj in range(KQ):
                    s_copy(npar, KQ + jj, rows_b, jj).wait()

            for jj in range(KQ):
                g_copy(par, KQ + jj, rows_b, jj).start()
            for jj in range(KQ):
                s_copy(par, jj, rows_a, jj).start(add=True)
            for jj in range(KQ):
                g_copy(par, KQ + jj, rows_b, jj).wait()
            for jj in range(KQ):
                s_copy(par, jj, rows_a, jj).wait()
            for jj in range(KQ):
                s_copy(par, KQ + jj, rows_b, jj).start(add=True)

            @pl.when(b < NB - 1)
            def _():
                load_src(b + 1, npar)
                load_dst(b + 1, npar)
                for jj in range(KQ):
                    g_copy(npar, jj, rows_a, jj).start()
            return carry2

        lax.fori_loop(0, NB, blk, 0)
        parl2 = lax.rem(NB - 1, 2)
        for jj in range(KQ):
            s_copy(parl2, KQ + jj, rows_b, jj).wait()
        plsc.subcore_barrier()
        pltpu.sync_copy(acc_slab, sums_out.at[cid, p].at[pl.ds(a0, ROWS_T)])
        return carry

    lax.fori_loop(0, CHUNKS, feat_pass, 0)


def _div_body(sums_ref, cnt_ref, hu_ref, hi_ref):
    outs = []
    for e in range(2):
        c = jnp.maximum(cnt_ref[e, :, 0:1], 1.0)
        h = jnp.concatenate([sums_ref[e, q] for q in range(CHUNKS)], axis=-1)
        outs.append(h / c)
    hi_ref[...] = outs[0]
    hu_ref[...] = outs[1]


def _divide(sums, cnt):
    return pl.pallas_call(
        _div_body,
        grid=(N // BN,),
        in_specs=[
            pl.BlockSpec((2, CHUNKS, BN, CW), lambda i: (0, 0, i, 0)),
            pl.BlockSpec((2, BN, CW), lambda i: (0, i, 0)),
        ],
        out_specs=[
            pl.BlockSpec((BN, D), lambda i: (i, 0)),
            pl.BlockSpec((BN, D), lambda i: (i, 0)),
        ],
        out_shape=[
            jax.ShapeDtypeStruct((N, D), jnp.float32),
            jax.ShapeDtypeStruct((N, D), jnp.float32),
        ],
    )(sums, cnt)


def kernel(feat_user, feat_item, W_rates, b_rates, W_rated_by, b_rated_by,
           edge_index_rates, edge_index_rated_by):
    ws = jnp.stack([W_rates, W_rated_by])
    bs = jnp.stack([b_rates, b_rated_by])
    pad_src = jnp.zeros((EP - E,), jnp.int32)
    pad_dst = jnp.full((EP - E,), N, jnp.int32)
    srcs = jnp.stack([
        jnp.concatenate([edge_index_rates[0], pad_src]),
        jnp.concatenate([edge_index_rated_by[0], pad_src]),
    ]).reshape(2, NS * RPT, GROUP)
    dsts = jnp.stack([
        jnp.concatenate([edge_index_rates[1], pad_dst]),
        jnp.concatenate([edge_index_rated_by[1], pad_dst]),
    ]).reshape(2, NS * RPT, GROUP)
    zslab = jnp.zeros((ROWS_T, CW), jnp.float32)
    ones_in = jnp.ones((GROUP, CW), jnp.float32)

    tabs = _make_tables(feat_user, feat_item, ws, bs)
    sums, cnt = _sc_segment_sums(tabs, srcs, dsts, zslab, ones_in)
    h_user, h_item = _divide(sums, cnt)
    return (h_user, h_item)


# speedup vs baseline: 3.7887x; 1.0220x over previous
import functools

import jax
import jax.numpy as jnp
from jax import lax
from jax.experimental import pallas as pl
from jax.experimental.pallas import tpu as pltpu
from jax.experimental.pallas import tpu_sc as plsc

N = 50000
D = 128
E = 600000
NC = 2
NS = 16
CHUNKS = 8
CW = 16
GROUP = 128
RPT = 296
EP = NS * RPT * GROUP
ACC_N = 50176
ROWS_T = ACC_N // NS
IDXB = 8
NB = RPT // IDXB
BN = 400


def _mm_body(fu_ref, fi_ref, w_ref, b_ref, wh_ref):
    for e in range(2):
        f = fu_ref[...] if e == 0 else fi_ref[...]
        wh = jnp.dot(f, w_ref[e], preferred_element_type=jnp.float32)
        wh_ref[e] = wh + b_ref[e][None, :]


def _make_wh(feat_user, feat_item, ws, bs):
    return pl.pallas_call(
        _mm_body,
        grid=(N // BN,),
        in_specs=[
            pl.BlockSpec((BN, D), lambda i: (i, 0)),
            pl.BlockSpec((BN, D), lambda i: (i, 0)),
            pl.BlockSpec((2, D, D), lambda i: (0, 0, 0)),
            pl.BlockSpec((2, D), lambda i: (0, 0)),
        ],
        out_specs=pl.BlockSpec((2, BN, D), lambda i: (0, i, 0)),
        out_shape=jax.ShapeDtypeStruct((2, N, D), jnp.float32),
    )(feat_user, feat_item, ws, bs)


_MESH = plsc.VectorSubcoreMesh(core_axis_name="c", subcore_axis_name="s")


@functools.partial(
    pl.kernel,
    out_type=(
        jax.ShapeDtypeStruct((2, ACC_N, CHUNKS, CW), jnp.float32),
        jax.ShapeDtypeStruct((2, ACC_N, CHUNKS, CW), jnp.float32),
    ),
    mesh=_MESH,
    compiler_params=pltpu.CompilerParams(use_tc_tiling_on_sc=False),
    scratch_types=[
        pltpu.VMEM((2, IDXB, GROUP), jnp.int32),
        pltpu.VMEM((2, IDXB, GROUP), jnp.int32),
        pltpu.VMEM((IDXB, GROUP, CW), jnp.float32),
        pltpu.VMEM((IDXB, GROUP, CW), jnp.float32),
        pltpu.VMEM((GROUP, CW), jnp.float32),
        pltpu.VMEM_SHARED((ACC_N, CW), jnp.float32),
        pltpu.SemaphoreType.DMA,
        pltpu.SemaphoreType.DMA,
    ],
)
def _sc_segment_sums(tabs, srcs8, dsts, zslab, ones_in,
                     sums_out, cnt_out,
                     src_idx, dst_idx, rows_a, rows_b, ones_v,
                     acc, gsem, ssem):
    cid = lax.axis_index("c")
    sid = lax.axis_index("s")
    r0 = sid * RPT
    a0 = sid * ROWS_T
    acc_slab = acc.at[pl.ds(a0, ROWS_T)]
    rows = (rows_a, rows_b)

    pltpu.sync_copy(ones_in, ones_v)

    def load_src(b, par, p):
        pltpu.sync_copy(srcs8.at[cid, pl.ds(r0 + b * IDXB, IDXB)],
                        src_idx.at[par])
        for j in range(IDXB):
            for k in range(GROUP // CW):
                sl = src_idx[par, j, pl.ds(k * CW, CW)]
                src_idx[par, j, pl.ds(k * CW, CW)] = sl + p

    def load_dst(b, par):
        pltpu.sync_copy(dsts.at[cid, pl.ds(r0 + b * IDXB, IDXB)],
                        dst_idx.at[par])

    pltpu.sync_copy(zslab, acc_slab)
    plsc.subcore_barrier()
    load_dst(0, 0)

    def cnt_blk(b, carry):
        par = lax.rem(b, 2)
        opar = lax.rem(b + 1, 2)
        for j in range(IDXB):
            pltpu.async_copy(ones_v, acc.at[dst_idx.at[par, j]], ssem,
                             add=True)

        @pl.when(b > 0)
        def _():
            for j in range(IDXB):
                pltpu.make_async_copy(ones_v, acc.at[dst_idx.at[opar, j]],
                                      ssem).wait()

        @pl.when(b < NB - 1)
        def _():
            load_dst(b + 1, opar)
        return carry

    lax.fori_loop(0, NB, cnt_blk, 0)
    parl = lax.rem(NB - 1, 2)
    for j in range(IDXB):
        pltpu.make_async_copy(ones_v, acc.at[dst_idx.at[parl, j]],
                              ssem).wait()
    plsc.subcore_barrier()
    for p8 in range(CHUNKS):
        pltpu.sync_copy(acc_slab, cnt_out.at[cid, pl.ds(a0, ROWS_T), p8])

    def feat_pass(p, carry):
        pltpu.sync_copy(zslab, acc_slab)
        plsc.subcore_barrier()
        tab = tabs.at[cid]

        def g_copy(par, j, rbuf):
            return pltpu.make_async_copy(tab.at[src_idx.at[par, j]],
                                         rbuf.at[j], gsem)

        def s_copy(par, j, rbuf):
            return pltpu.make_async_copy(rbuf.at[j],
                                         acc.at[dst_idx.at[par, j]], ssem)

        load_src(0, 0, p)
        load_dst(0, 0)
        for j in range(IDXB):
            g_copy(0, j, rows_a).start()

        def blk(b, carry2):
            par = lax.rem(b, 2)
            for ri in range(2):
                rbuf = rows[ri]
                obuf = rows[1 - ri]

                @pl.when(par == ri)
                def _(b=b, ri=ri, rbuf=rbuf, obuf=obuf):
                    for j in range(IDXB):
                        g_copy(ri, j, rbuf).wait()

                    @pl.when(b > 0)
                    def _():
                        for j in range(IDXB):
                            s_copy(1 - ri, j, obuf).wait()

                    for j in range(IDXB):
                        s_copy(ri, j, rbuf).start(add=True)

                    @pl.when(b < NB - 1)
                    def _():
                        load_src(b + 1, 1 - ri, p)
                        load_dst(b + 1, 1 - ri)
                        for j in range(IDXB):
                            g_copy(1 - ri, j, obuf).start()
            return carry2

        lax.fori_loop(0, NB, blk, 0)
        parl2 = lax.rem(NB - 1, 2)
        for ri in range(2):
            @pl.when(parl2 == ri)
            def _(ri=ri):
                for j in range(IDXB):
                    s_copy(ri, j, rows[ri]).wait()
        plsc.subcore_barrier()
        pltpu.sync_copy(acc_slab, sums_out.at[cid, pl.ds(a0, ROWS_T), p])
        return carry

    lax.fori_loop(0, CHUNKS, feat_pass, 0)


def _div_body(sums_ref, cnt_ref, hu_ref, hi_ref):
    hi_ref[...] = sums_ref[0] / jnp.maximum(cnt_ref[0], 1.0)
    hu_ref[...] = sums_ref[1] / jnp.maximum(cnt_ref[1], 1.0)


def _divide(sums, cnt):
    return pl.pallas_call(
        _div_body,
        grid=(N // BN,),
        in_specs=[
            pl.BlockSpec((2, BN, D), lambda i: (0, i, 0)),
            pl.BlockSpec((2, BN, D), lambda i: (0, i, 0)),
        ],
        out_specs=[
            pl.BlockSpec((BN, D), lambda i: (i, 0)),
            pl.BlockSpec((BN, D), lambda i: (i, 0)),
        ],
        out_shape=[
            jax.ShapeDtypeStruct((N, D), jnp.float32),
            jax.ShapeDtypeStruct((N, D), jnp.float32),
        ],
    )(sums, cnt)


def kernel(feat_user, feat_item, W_rates, b_rates, W_rated_by, b_rated_by,
           edge_index_rates, edge_index_rated_by):
    ws = jnp.stack([W_rates, W_rated_by])
    bs = jnp.stack([b_rates, b_rated_by])
    pad = jnp.zeros((EP - E,), jnp.int32)
    pad_dst = jnp.full((EP - E,), N, jnp.int32)
    srcs8 = jnp.stack([
        jnp.concatenate([edge_index_rates[0] * 8, pad]),
        jnp.concatenate([edge_index_rated_by[0] * 8, pad]),
    ]).reshape(2, NS * RPT, GROUP)
    dsts = jnp.stack([
        jnp.concatenate([edge_index_rates[1], pad_dst]),
        jnp.concatenate([edge_index_rated_by[1], pad_dst]),
    ]).reshape(2, NS * RPT, GROUP)
    zslab = jnp.zeros((ROWS_T, CW), jnp.float32)
    ones_in = jnp.ones((GROUP, CW), jnp.float32)

    wh = _make_wh(feat_user, feat_item, ws, bs)
    tabs = wh.reshape(2, N * CHUNKS, CW)
    sums, cnt = _sc_segment_sums(tabs, srcs8, dsts, zslab, ones_in)
    sums = sums.reshape(2, ACC_N, D)
    cnt = cnt.reshape(2, ACC_N, D)
    h_item, h_user = _divide(sums, cnt)
    return (h_user, h_item)
